# Initial kernel scaffold; baseline (speedup 1.0000x reference)
import jax
import jax.numpy as jnp
from jax.experimental import pallas as pl


def kernel(vertices, centroids, centroid_features, W1, b1, W2, b2):
    raise NotImplementedError("write your pallas kernel here")


# speedup vs baseline: 45.5081x; 45.5081x over previous
import functools

import jax
import jax.numpy as jnp
from jax.experimental import pallas as pl

N = 16384
M = 1024
C = 128
H = 256
TN = 512


def _fused_body(v_ref, ct_ref, f_ref, w1_ref, b1_ref, w2_ref, b2_ref, o_ref):
    v = v_ref[...]
    ct = ct_ref[...]

    d2 = (v[:, 0:1] - ct[0:1, :]) ** 2
    d2 += (v[:, 1:2] - ct[1:2, :]) ** 2
    d2 += (v[:, 2:3] - ct[2:3, :]) ** 2

    inf = jnp.float32(jnp.inf)
    m0 = jnp.min(d2, axis=1, keepdims=True)
    d2a = jnp.where(d2 > m0, d2, inf)
    m1 = jnp.min(d2a, axis=1, keepdims=True)
    d2b = jnp.where(d2a > m1, d2a, inf)
    m2 = jnp.min(d2b, axis=1, keepdims=True)

    recip = 1.0 / d2
    wmat = jnp.where(d2 <= m2, recip, 0.0)

    has_inf = jnp.isinf(1.0 / m0)
    exact = (d2 <= m0).astype(jnp.float32)
    wmat = jnp.where(has_inf, exact, wmat)

    wsum = jnp.sum(wmat, axis=1, keepdims=True)
    interp = jnp.dot(wmat, f_ref[...], preferred_element_type=jnp.float32)
    interp = interp / wsum

    h = jnp.dot(interp, w1_ref[...], preferred_element_type=jnp.float32)
    h = jnp.maximum(h + b1_ref[...], 0.0)
    out = jnp.dot(h, w2_ref[...], preferred_element_type=jnp.float32)
    o_ref[...] = out + b2_ref[...]


@functools.partial(jax.jit, static_argnames=())
def kernel(vertices, centroids, centroid_features, W1, b1, W2, b2):
    ct = centroids.T
    w1a = W1[:C, :]
    b1r = b1.reshape(1, H)
    b2r = b2.reshape(1, H)
    grid = (N // TN,)
    out = pl.pallas_call(
        _fused_body,
        grid=grid,
        in_specs=[
            pl.BlockSpec((TN, 3), lambda i: (i, 0)),
            pl.BlockSpec((3, M), lambda i: (0, 0)),
            pl.BlockSpec((M, C), lambda i: (0, 0)),
            pl.BlockSpec((C, H), lambda i: (0, 0)),
            pl.BlockSpec((1, H), lambda i: (0, 0)),
            pl.BlockSpec((H, H), lambda i: (0, 0)),
            pl.BlockSpec((1, H), lambda i: (0, 0)),
        ],
        out_specs=pl.BlockSpec((TN, H), lambda i: (i, 0)),
        out_shape=jax.ShapeDtypeStruct((N, H), jnp.float32),
    )(vertices, ct, centroid_features, w1a, b1r, W2, b2r)
    return out


# speedup vs baseline: 55.2958x; 1.2151x over previous
import functools

import jax
import jax.numpy as jnp
from jax.experimental import pallas as pl

N = 16384
M = 1024
C = 128
H = 256
TN = 512


def _fused_body(v_ref, ct_ref, f_ref, w1_ref, b1_ref, w2_ref, b2_ref, o_ref):
    v = v_ref[...]
    ct = ct_ref[...]

    vv = jnp.sum(v * v, axis=1, keepdims=True)
    ones_col = jnp.ones_like(vv)
    a = jnp.concatenate([v * -2.0, vv, ones_col], axis=1)
    c2 = jnp.sum(ct * ct, axis=0, keepdims=True)
    b = jnp.concatenate([ct, jnp.ones_like(c2), c2], axis=0)
    d2 = jnp.dot(a, b, preferred_element_type=jnp.float32)
    d2 = jnp.maximum(d2, 0.0)

    inf = jnp.float32(jnp.inf)
    m0 = jnp.min(d2, axis=1, keepdims=True)
    d2a = jnp.where(d2 > m0, d2, inf)
    m1 = jnp.min(d2a, axis=1, keepdims=True)
    d2b = jnp.where(d2a > m1, d2a, inf)
    m2 = jnp.min(d2b, axis=1, keepdims=True)

    w0 = 1.0 / m0
    has_inf = jnp.isinf(w0)
    thresh = jnp.where(has_inf, m0, m2)
    wsum = jnp.where(has_inf, 1.0, w0 + 1.0 / m1 + 1.0 / m2)

    recip = 1.0 / d2
    val = jnp.where(has_inf, 1.0, recip)
    wmat = jnp.where(d2 <= thresh, val, 0.0)

    interp = jnp.dot(wmat, f_ref[...], preferred_element_type=jnp.float32)
    interp = interp * (1.0 / wsum)

    h = jnp.dot(interp, w1_ref[...], preferred_element_type=jnp.float32)
    h = jnp.maximum(h + b1_ref[...], 0.0)
    out = jnp.dot(h, w2_ref[...], preferred_element_type=jnp.float32)
    o_ref[...] = out + b2_ref[...]


@functools.partial(jax.jit, static_argnames=())
def kernel(vertices, centroids, centroid_features, W1, b1, W2, b2):
    ct = centroids.T
    w1a = W1[:C, :]
    b1r = b1.reshape(1, H)
    b2r = b2.reshape(1, H)
    grid = (N // TN,)
    out = pl.pallas_call(
        _fused_body,
        grid=grid,
        in_specs=[
            pl.BlockSpec((TN, 3), lambda i: (i, 0)),
            pl.BlockSpec((3, M), lambda i: (0, 0)),
            pl.BlockSpec((M, C), lambda i: (0, 0)),
            pl.BlockSpec((C, H), lambda i: (0, 0)),
            pl.BlockSpec((1, H), lambda i: (0, 0)),
            pl.BlockSpec((H, H), lambda i: (0, 0)),
            pl.BlockSpec((1, H), lambda i: (0, 0)),
        ],
        out_specs=pl.BlockSpec((TN, H), lambda i: (i, 0)),
        out_shape=jax.ShapeDtypeStruct((N, H), jnp.float32),
    )(vertices, ct, centroid_features, w1a, b1r, W2, b2r)
    return out
